# baseline (speedup 1.0000x reference)
import functools

import jax
import jax.numpy as jnp
from jax import lax
from jax.experimental import pallas as pl
from jax.experimental.pallas import tpu as pltpu
from jax.experimental.pallas import tpu_sc as plsc

_K1 = [21, 15, 9, 5]
_P1 = [10, 7, 4, 2]
_K2 = [9, 7, 5, 3]
_P2 = [4, 3, 2, 1]
_GROUPS = 4

_V = 8192
_D = 200
_DP = 256
_DA = 256
_M = 8192
_TM = 256
_NT = _M // _TM


def _conv1d(x, w, b, pad):
    y = lax.conv_general_dilated(
        x, w, window_strides=(1, 1), padding=((0, 0), (pad, pad)),
        dimension_numbers=('NCHW', 'OIHW', 'NCHW'))
    return y + b[None, :, None, None]


def _groupnorm(x, g, b, groups=_GROUPS, eps=1e-5):
    B, C, H, W = x.shape
    xg = x.reshape(B, groups, C // groups, H, W)
    mu = xg.mean(axis=(2, 3, 4), keepdims=True)
    var = xg.var(axis=(2, 3, 4), keepdims=True)
    xg = (xg - mu) / jnp.sqrt(var + eps)
    xn = xg.reshape(B, C, H, W)
    return xn * g[None, :, None, None] + b[None, :, None, None]


def _pool(x, k):
    B, C, H, W = x.shape
    return x.reshape(B, C, H, W // k, k).mean(axis=-1)


def _branch(x, i, p):
    h = _pool(jax.nn.gelu(_groupnorm(
        _conv1d(x, p['c1w'][i], p['c1b'][i], _P1[i]),
        p['g1w'][i], p['g1b'][i]), approximate=False), 2)
    h = _pool(jax.nn.gelu(_groupnorm(
        _conv1d(h, p['c2w'][i], p['c2b'][i], _P2[i]),
        p['g2w'][i], p['g2b'][i]), approximate=False), 4)
    B, C, NA, T = h.shape
    return jnp.transpose(h, (0, 2, 3, 1)).reshape(B, NA, T * C)


_PTK = 2048


def _pad_body(cb_ref, out_ref):
    out_ref[...] = jnp.pad(cb_ref[...], ((0, 0), (0, 0), (0, _DP - _D)))


def _pad_cb(cb):
    return pl.pallas_call(
        _pad_body,
        grid=(2, _V // _PTK),
        in_specs=[pl.BlockSpec((1, _PTK, _D), lambda l, j: (l, j, 0))],
        out_specs=pl.BlockSpec((1, _PTK, _DP), lambda l, j: (l, j, 0)),
        out_shape=jax.ShapeDtypeStruct((2, _V, _DP), jnp.float32),
    )(cb)


def _dist_body(r_ref, rn_ref, cb_ref, cn_ref, idx_ref):
    dot = lax.dot_general(r_ref[...], cb_ref[...], (((1,), (1,)), ((), ())),
                          preferred_element_type=jnp.float32)
    d = (rn_ref[...] - 2.0 * dot) + cn_ref[...]
    m = jnp.min(d, axis=1, keepdims=True)
    ii = lax.broadcasted_iota(jnp.int32, d.shape, 1)
    idx = jnp.min(jnp.where(d == m, ii, jnp.int32(_V)), axis=1)
    idx_ref[...] = idx.reshape(1, 1, _TM)


_TOK_SPEC = pl.BlockSpec((_TM, _D), lambda i: (i, 0))
_RN_SPEC = pl.BlockSpec((_TM, 1), lambda i: (i, 0))
_CB_SPEC = pl.BlockSpec((_V, _D), lambda i: (0, 0))
_CN_SPEC = pl.BlockSpec((1, _V), lambda i: (0, 0))
_IDX_SPEC = pl.BlockSpec((1, 1, _TM), lambda i: (i, 0, 0))
_IDX_SHAPE = jax.ShapeDtypeStruct((_NT, 1, _TM), jnp.int32)


def _nearest(r_pad, rnorm, cb_l, cnorm_l):
    return pl.pallas_call(
        _dist_body,
        grid=(_NT,),
        in_specs=[_TOK_SPEC, _RN_SPEC, _CB_SPEC, _CN_SPEC],
        out_specs=_IDX_SPEC,
        out_shape=_IDX_SHAPE,
    )(r_pad, rnorm, cb_l, cnorm_l).reshape(_M)


def _gather_rows(table, idx):
    info = plsc.get_sparse_core_info()
    nw = info.num_cores * info.num_subcores
    bpw = _M // nw
    mesh = plsc.VectorSubcoreMesh(core_axis_name="c", subcore_axis_name="s")

    @functools.partial(
        pl.kernel, mesh=mesh,
        out_type=jax.ShapeDtypeStruct((_M, _DP), jnp.float32),
        scratch_types=[
            pltpu.VMEM((bpw,), jnp.int32),
            pltpu.VMEM((bpw, _DP), jnp.float32),
            pltpu.SemaphoreType.DMA,
        ],
    )
    def k(table_hbm, idx_hbm, out_hbm, idx_v, rows_v, sem):
        wid = lax.axis_index("s") * info.num_cores + lax.axis_index("c")
        base = wid * bpw
        pltpu.sync_copy(idx_hbm.at[pl.ds(base, bpw)], idx_v)
        pltpu.async_copy(table_hbm.at[idx_v], rows_v, sem).wait()
        pltpu.sync_copy(rows_v, out_hbm.at[pl.ds(base, bpw)])

    return k(table, idx)


def kernel(x, params):
    p = params
    B, N, A, T = x.shape
    h = x.reshape(B, N * A, T)[:, None, :, :]
    zs = [_branch(h, i, p) for i in range(4)]
    zf = jnp.concatenate([z.reshape(-1, _D) for z in zs], axis=0)

    cb = p['codebooks']
    cb_pad = _pad_cb(cb)
    cnorm = (cb ** 2).sum(-1)[:, None, :]

    rn0 = (zf ** 2).sum(-1, keepdims=True)
    idx0 = _nearest(zf, rn0, cb[0], cnorm[0])
    q0 = _gather_rows(cb_pad[0], idx0)[:, :_D]

    r1 = zf - q0
    rn1 = (r1 ** 2).sum(-1, keepdims=True)
    idx1 = _nearest(r1, rn1, cb[1], cnorm[1])
    q1 = _gather_rows(cb_pad[1], idx1)[:, :_D]

    total = q0 + q1
    out = zf + (total - zf)
    return out.reshape(4, B, N * A, _D)


# speedup vs baseline: 1.1028x; 1.1028x over previous
import functools

import jax
import jax.numpy as jnp
from jax import lax
from jax.experimental import pallas as pl
from jax.experimental.pallas import tpu as pltpu
from jax.experimental.pallas import tpu_sc as plsc

_K1 = [21, 15, 9, 5]
_P1 = [10, 7, 4, 2]
_K2 = [9, 7, 5, 3]
_P2 = [4, 3, 2, 1]
_GROUPS = 4

_V = 8192
_D = 200
_DP = 256
_DA = 256
_M = 8192
_TM = 256
_NT = _M // _TM


def _conv1d(x, w, b, pad):
    y = lax.conv_general_dilated(
        x, w, window_strides=(1, 1), padding=((0, 0), (pad, pad)),
        dimension_numbers=('NCHW', 'OIHW', 'NCHW'))
    return y + b[None, :, None, None]


def _groupnorm(x, g, b, groups=_GROUPS, eps=1e-5):
    B, C, H, W = x.shape
    xg = x.reshape(B, groups, C // groups, H, W)
    mu = xg.mean(axis=(2, 3, 4), keepdims=True)
    var = xg.var(axis=(2, 3, 4), keepdims=True)
    xg = (xg - mu) / jnp.sqrt(var + eps)
    xn = xg.reshape(B, C, H, W)
    return xn * g[None, :, None, None] + b[None, :, None, None]


def _pool(x, k):
    B, C, H, W = x.shape
    return x.reshape(B, C, H, W // k, k).mean(axis=-1)


def _branch(x, i, p):
    h = _pool(jax.nn.gelu(_groupnorm(
        _conv1d(x, p['c1w'][i], p['c1b'][i], _P1[i]),
        p['g1w'][i], p['g1b'][i]), approximate=False), 2)
    h = _pool(jax.nn.gelu(_groupnorm(
        _conv1d(h, p['c2w'][i], p['c2b'][i], _P2[i]),
        p['g2w'][i], p['g2b'][i]), approximate=False), 4)
    B, C, NA, T = h.shape
    return jnp.transpose(h, (0, 2, 3, 1)).reshape(B, NA, T * C)


_KMAX1, _PMAX1 = 21, 10
_KMAX2, _PMAX2 = 9, 4


def _padw(w, kmax, pad, pmax):
    wz = jnp.zeros(w.shape[:-1] + (kmax,), w.dtype)
    return lax.dynamic_update_slice(wz, w, (0, 0, 0, pmax - pad))


def _gn16(xx, g, b, eps=1e-5):
    B, C, H, W = xx.shape
    xg = xx.reshape(B, 16, 2, H, W)
    mu = xg.mean(axis=(2, 3, 4), keepdims=True)
    var = xg.var(axis=(2, 3, 4), keepdims=True)
    xg = (xg - mu) / jnp.sqrt(var + eps)
    return (xg.reshape(B, C, H, W) * g[None, :, None, None]
            + b[None, :, None, None])


def _front_end(h, p):
    B = h.shape[0]
    NA = h.shape[2]
    w1 = jnp.concatenate([_padw(p['c1w'][i], _KMAX1, _P1[i], _PMAX1)
                          for i in range(4)], axis=0)
    b1 = jnp.concatenate([p['c1b'][i] for i in range(4)])
    w2 = jnp.concatenate([_padw(p['c2w'][i], _KMAX2, _P2[i], _PMAX2)
                          for i in range(4)], axis=0)
    b2 = jnp.concatenate([p['c2b'][i] for i in range(4)])
    g1 = jnp.concatenate([p['g1w'][i] for i in range(4)])
    gb1 = jnp.concatenate([p['g1b'][i] for i in range(4)])
    g2 = jnp.concatenate([p['g2w'][i] for i in range(4)])
    gb2 = jnp.concatenate([p['g2b'][i] for i in range(4)])

    hrep = jnp.broadcast_to(h, (B, 4) + h.shape[2:])
    y = lax.conv_general_dilated(hrep, w1, window_strides=(1, 1),
                                 padding=((0, 0), (_PMAX1, _PMAX1)),
                                 dimension_numbers=('NCHW', 'OIHW', 'NCHW'),
                                 feature_group_count=4)
    y = y + b1[None, :, None, None]
    y = _pool(jax.nn.gelu(_gn16(y, g1, gb1), approximate=False), 2)
    y = lax.conv_general_dilated(y, w2, window_strides=(1, 1),
                                 padding=((0, 0), (_PMAX2, _PMAX2)),
                                 dimension_numbers=('NCHW', 'OIHW', 'NCHW'),
                                 feature_group_count=4)
    y = y + b2[None, :, None, None]
    y = _pool(jax.nn.gelu(_gn16(y, g2, gb2), approximate=False), 4)
    return jnp.concatenate(
        [jnp.transpose(y[:, 8 * i:8 * (i + 1)], (0, 2, 3, 1)).reshape(-1, _D)
         for i in range(4)], axis=0)


_PTK = 2048


def _pad_body(cb_ref, out_ref):
    out_ref[...] = jnp.pad(cb_ref[...], ((0, 0), (0, 0), (0, _DP - _D)))


def _pad_cb(cb):
    return pl.pallas_call(
        _pad_body,
        grid=(2, _V // _PTK),
        in_specs=[pl.BlockSpec((1, _PTK, _D), lambda l, j: (l, j, 0))],
        out_specs=pl.BlockSpec((1, _PTK, _DP), lambda l, j: (l, j, 0)),
        out_shape=jax.ShapeDtypeStruct((2, _V, _DP), jnp.float32),
    )(cb)


def _dist_body(r_ref, rn_ref, cb_ref, cn_ref, idx_ref):
    dot = lax.dot_general(r_ref[...], cb_ref[...], (((1,), (1,)), ((), ())),
                          preferred_element_type=jnp.float32)
    d = (rn_ref[...] - 2.0 * dot) + cn_ref[...]
    m = jnp.min(d, axis=1, keepdims=True)
    ii = lax.broadcasted_iota(jnp.int32, d.shape, 1)
    idx = jnp.min(jnp.where(d == m, ii, jnp.int32(_V)), axis=1)
    idx_ref[...] = idx.reshape(1, 1, _TM)


_TOK_SPEC = pl.BlockSpec((_TM, _D), lambda i: (i, 0))
_RN_SPEC = pl.BlockSpec((_TM, 1), lambda i: (i, 0))
_CB_SPEC = pl.BlockSpec((_V, _D), lambda i: (0, 0))
_CN_SPEC = pl.BlockSpec((1, _V), lambda i: (0, 0))
_IDX_SPEC = pl.BlockSpec((1, 1, _TM), lambda i: (i, 0, 0))
_IDX_SHAPE = jax.ShapeDtypeStruct((_NT, 1, _TM), jnp.int32)


def _nearest(r_pad, rnorm, cb_l, cnorm_l):
    return pl.pallas_call(
        _dist_body,
        grid=(_NT,),
        in_specs=[_TOK_SPEC, _RN_SPEC, _CB_SPEC, _CN_SPEC],
        out_specs=_IDX_SPEC,
        out_shape=_IDX_SHAPE,
    )(r_pad, rnorm, cb_l, cnorm_l).reshape(_M)


def _gather_rows(table, idx):
    info = plsc.get_sparse_core_info()
    nw = info.num_cores * info.num_subcores
    bpw = _M // nw
    mesh = plsc.VectorSubcoreMesh(core_axis_name="c", subcore_axis_name="s")

    @functools.partial(
        pl.kernel, mesh=mesh,
        out_type=jax.ShapeDtypeStruct((_M, _DP), jnp.float32),
        scratch_types=[
            pltpu.VMEM((bpw,), jnp.int32),
            pltpu.VMEM((bpw, _DP), jnp.float32),
            pltpu.SemaphoreType.DMA,
        ],
    )
    def k(table_hbm, idx_hbm, out_hbm, idx_v, rows_v, sem):
        wid = lax.axis_index("s") * info.num_cores + lax.axis_index("c")
        base = wid * bpw
        pltpu.sync_copy(idx_hbm.at[pl.ds(base, bpw)], idx_v)
        pltpu.async_copy(table_hbm.at[idx_v], rows_v, sem).wait()
        pltpu.sync_copy(rows_v, out_hbm.at[pl.ds(base, bpw)])

    return k(table, idx)


def kernel(x, params):
    p = params
    B, N, A, T = x.shape
    h = x.reshape(B, N * A, T)[:, None, :, :]
    zf = _front_end(h, p)

    cb = p['codebooks']
    cb_pad = _pad_cb(cb)
    cnorm = (cb ** 2).sum(-1)[:, None, :]

    rn0 = (zf ** 2).sum(-1, keepdims=True)
    idx0 = _nearest(zf, rn0, cb[0], cnorm[0])
    q0 = _gather_rows(cb_pad[0], idx0)[:, :_D]

    r1 = zf - q0
    rn1 = (r1 ** 2).sum(-1, keepdims=True)
    idx1 = _nearest(r1, rn1, cb[1], cnorm[1])
    q1 = _gather_rows(cb_pad[1], idx1)[:, :_D]

    total = q0 + q1
    out = zf + (total - zf)
    return out.reshape(4, B, N * A, _D)
